# Initial kernel scaffold; baseline (speedup 1.0000x reference)
import jax
import jax.numpy as jnp
from jax.experimental import pallas as pl


def kernel(means3d, opacities, cov3D, radii, features):
    raise NotImplementedError("write your pallas kernel here")


# speedup vs baseline: 16.1139x; 16.1139x over previous
import functools
import numpy as np
import jax
import jax.numpy as jnp
from jax import lax
from jax.experimental import pallas as pl
from jax.experimental.pallas import tpu as pltpu
from jax.experimental.pallas import tpu_sc as plsc

N_GAUSS = 50000
N_PAD = 50176
GRID = 50
NV = GRID * GRID * GRID
NVP = 126976
ROWS_PER_TILE = NVP // 16
ZCHUNK = ROWS_PER_TILE // 4
NCH = 9
PC = 48
W_PAD = 384
G_PER_TILE = N_PAD // 16
CHUNK_G = 112
N_CHUNKS = G_PER_TILE // CHUNK_G
CLASS_GROUPS = (1, 2, 8, 24)


def _make_offset_planes():
    r = np.arange(-3, 4)
    ox, oy, oz = np.meshgrid(r, r, r, indexing='ij')
    ox, oy, oz = ox.ravel(), oy.ravel(), oz.ravel()
    cheb = np.max(np.abs(np.stack([ox, oy, oz])), axis=0)
    order = np.argsort(cheb, kind='stable')
    ox, oy, oz, cheb = ox[order], oy[order], oz[order], cheb[order]
    flatoff = ox * (GRID * GRID) + oy * GRID + oz
    planes = np.zeros((5, W_PAD), np.float32)
    planes[3, :] = 1e9
    planes[0, :343] = ox
    planes[1, :343] = oy
    planes[2, :343] = oz
    planes[3, :343] = cheb
    planes[4, :343] = flatoff
    return planes.reshape(-1)


_OFFSET_PLANES = _make_offset_planes()


def _voxelize_body(par_hbm, offs_hbm, zrow_hbm, out0, out1,
                   offs_v, par_v, zbuf, ib16, ib32, ib128,
                   vt16, vt32, vt128, accs):
    cid = lax.axis_index("c")
    sid = lax.axis_index("s")
    is0 = cid == 0

    pltpu.sync_copy(offs_hbm, offs_v)

    base_row = sid * ROWS_PER_TILE
    pltpu.sync_copy(zrow_hbm, zbuf)
    for j in range(NCH):
        for z in range(4):
            pltpu.sync_copy(
                zbuf, accs[j].at[pl.ds(base_row + z * ZCHUNK, ZCHUNK)])
    plsc.subcore_barrier()

    def do_gauss(gi, _):
        pa = par_v[gi, pl.ds(0, 16)]
        chv0 = par_v[gi, pl.ds(16, 16)]
        chv1 = par_v[gi, pl.ds(32, 16)]
        chvec = jnp.where(is0, chv0, chv1)
        mx = pa[0]
        my = pa[1]
        mz = pa[2]
        rf = pa[3]
        c0 = pa[4]
        c1 = pa[5]
        c2 = pa[6]
        c3 = pa[7]
        c4 = pa[8]
        c5 = pa[9]
        cx = pa[10]
        cy = pa[11]
        cz = pa[12]
        flatc = pa[13]
        bx = -5.0 - mx
        by = -5.0 - my
        bz = -5.0 - mz
        ch = [chvec[j] for j in range(NCH)]

        def emit_groups(k0, ng, ib, vt):
            for kk in range(ng):
                o = (k0 + kk) * 16
                oxv = offs_v[pl.ds(o, 16)]
                oyv = offs_v[pl.ds(W_PAD + o, 16)]
                ozv = offs_v[pl.ds(2 * W_PAD + o, 16)]
                chbv = offs_v[pl.ds(3 * W_PAD + o, 16)]
                fov = offs_v[pl.ds(4 * W_PAD + o, 16)]
                vx = cx + oxv
                vy = cy + oyv
                vz = cz + ozv
                valid = ((chbv <= rf)
                         & (vx >= 0.0) & (vx <= 49.0)
                         & (vy >= 0.0) & (vy <= 49.0)
                         & (vz >= 0.0) & (vz <= 49.0))
                dx = (vx + 0.5) * 0.2 + bx
                dy = (vy + 0.5) * 0.2 + by
                dz = (vz + 0.5) * 0.2 + bz
                pw = (-0.5 * (c0 * dx * dx + c1 * dy * dy + c2 * dz * dz)
                      - (c3 * dx * dy + c4 * dy * dz + c5 * dx * dz))
                w = jnp.where(valid, jnp.exp(pw), 0.0)
                fl = jnp.where(valid, (flatc + fov).astype(jnp.int32), 0)
                ib[pl.ds(kk * 16, 16)] = fl
                for j in range(NCH):
                    vt[j, pl.ds(kk * 16, 16)] = w * ch[j]

        def fire(ib, vt):
            for j in range(8):
                pltpu.sync_copy(vt.at[j], accs[j].at[ib], add=True)

            @pl.when(is0)
            def _():
                pltpu.sync_copy(vt.at[8], accs[8].at[ib], add=True)

        @pl.when(rf == 0.0)
        def _():
            emit_groups(0, 1, ib16, vt16)
            fire(ib16, vt16)

        @pl.when(rf == 1.0)
        def _():
            emit_groups(0, 2, ib32, vt32)
            fire(ib32, vt32)

        @pl.when(rf == 2.0)
        def _():
            emit_groups(0, 8, ib128, vt128)
            fire(ib128, vt128)

        @pl.when(rf == 3.0)
        def _():
            for seg in range(3):
                emit_groups(seg * 8, 8, ib128, vt128)
                fire(ib128, vt128)

        return 0

    def do_chunk(chk, _):
        g0 = sid * G_PER_TILE + chk * CHUNK_G
        pltpu.sync_copy(par_hbm.at[pl.ds(g0, CHUNK_G)], par_v)
        lax.fori_loop(0, CHUNK_G, do_gauss, 0)
        return 0

    lax.fori_loop(0, N_CHUNKS, do_chunk, 0)

    plsc.subcore_barrier()

    for j in range(NCH):
        for z in range(4):
            sl = pl.ds(base_row + z * ZCHUNK, ZCHUNK)
            pltpu.sync_copy(accs[j].at[sl], zbuf)

            @pl.when(is0)
            def _():
                pltpu.sync_copy(zbuf, out0.at[j, sl])

            @pl.when(jnp.logical_not(is0))
            def _():
                pltpu.sync_copy(zbuf, out1.at[j, sl])


@functools.partial(
    pl.kernel,
    out_type=(jax.ShapeDtypeStruct((NCH, NVP), jnp.float32),
              jax.ShapeDtypeStruct((NCH, NVP), jnp.float32)),
    mesh=plsc.VectorSubcoreMesh(core_axis_name="c", subcore_axis_name="s",
                                num_cores=2, num_subcores=16),
    scratch_types=[
        pltpu.VMEM((5 * W_PAD,), jnp.float32),
        pltpu.VMEM((CHUNK_G, PC), jnp.float32),
        pltpu.VMEM((ZCHUNK,), jnp.float32),
        pltpu.VMEM((16,), jnp.int32),
        pltpu.VMEM((32,), jnp.int32),
        pltpu.VMEM((128,), jnp.int32),
        pltpu.VMEM((NCH, 16), jnp.float32),
        pltpu.VMEM((NCH, 32), jnp.float32),
        pltpu.VMEM((NCH, 128), jnp.float32),
    ] + [pltpu.VMEM_SHARED((NVP,), jnp.float32) for _ in range(NCH)],
    compiler_params=pltpu.CompilerParams(use_tc_tiling_on_sc=False),
)
def _voxelize(par_hbm, offs_hbm, zrow_hbm, out0, out1, *scratch):
    _voxelize_body(par_hbm, offs_hbm, zrow_hbm, out0, out1,
                   *scratch[:9], scratch[9:])


@jax.jit
def kernel(means3d, opacities, cov3D, radii, features):
    n = means3d.shape[0]
    ones1 = jnp.ones((n, 1), jnp.float32)
    zeros7 = jnp.zeros((n, 7), jnp.float32)
    zeros8 = jnp.zeros((n, 8), jnp.float32)
    chv0 = opacities * jnp.concatenate([ones1, features[:, :8], zeros7],
                                       axis=1)
    chv1 = opacities * jnp.concatenate([features[:, 8:], zeros8], axis=1)
    center = jnp.floor((means3d - (-5.0)) / 0.2)
    flatc = (center[:, :1] * 2500.0 + center[:, 1:2] * 50.0
             + center[:, 2:3])
    params = jnp.concatenate(
        [means3d, radii.astype(jnp.float32)[:, None], cov3D,
         center, flatc, jnp.zeros((n, 2), jnp.float32),
         chv0, chv1], axis=1)
    params = jnp.concatenate(
        [params, jnp.zeros((N_PAD - n, PC), jnp.float32)], axis=0)
    offs = jnp.asarray(_OFFSET_PLANES)
    zrow = jnp.zeros((ZCHUNK,), jnp.float32)
    out0, out1 = _voxelize(params, offs, zrow)
    acc = jnp.concatenate([out0[:, :NV].T, out1[:8, :NV].T],
                          axis=1)
    grid_density = acc[:, :1].reshape(GRID, GRID, GRID, 1)
    grid_feats = acc[:, 1:].reshape(GRID, GRID, GRID, 16)
    return grid_density, grid_feats


# speedup vs baseline: 16.8478x; 1.0455x over previous
import functools
import numpy as np
import jax
import jax.numpy as jnp
from jax import lax
from jax.experimental import pallas as pl
from jax.experimental.pallas import tpu as pltpu
from jax.experimental.pallas import tpu_sc as plsc

N_GAUSS = 50000
N_PAD = 50176
GRID = 50
NV = GRID * GRID * GRID
NVP = 126976
ROWS_PER_TILE = NVP // 16
ZCHUNK = ROWS_PER_TILE // 4
NCH = 9
PC = 48
W_PAD = 384
G_PER_TILE = N_PAD // 16
CHUNK_G = 112
N_CHUNKS = G_PER_TILE // CHUNK_G
CLASS_GROUPS = (1, 2, 8, 24)


def _make_offset_planes():
    r = np.arange(-3, 4)
    ox, oy, oz = np.meshgrid(r, r, r, indexing='ij')
    ox, oy, oz = ox.ravel(), oy.ravel(), oz.ravel()
    cheb = np.max(np.abs(np.stack([ox, oy, oz])), axis=0)
    order = np.argsort(cheb, kind='stable')
    ox, oy, oz, cheb = ox[order], oy[order], oz[order], cheb[order]
    flatoff = ox * (GRID * GRID) + oy * GRID + oz
    planes = np.zeros((5, W_PAD), np.float32)
    planes[3, :] = 1e9
    planes[0, :343] = ox
    planes[1, :343] = oy
    planes[2, :343] = oz
    planes[3, :343] = cheb
    planes[4, :343] = flatoff
    return planes.reshape(-1)


_OFFSET_PLANES = _make_offset_planes()


def _voxelize_body(par_hbm, offs_hbm, zrow_hbm, out0, out1,
                   offs_v, par_v, zbuf, ib16, ib32, ib128,
                   vt16, vt32, vt128, sem, accs):
    cid = lax.axis_index("c")
    sid = lax.axis_index("s")
    is0 = cid == 0

    pltpu.sync_copy(offs_hbm, offs_v)

    base_row = sid * ROWS_PER_TILE
    pltpu.sync_copy(zrow_hbm, zbuf)
    for j in range(NCH):
        for z in range(4):
            pltpu.sync_copy(
                zbuf, accs[j].at[pl.ds(base_row + z * ZCHUNK, ZCHUNK)])
    plsc.subcore_barrier()

    def do_gauss(gi, _):
        pa = par_v[gi, pl.ds(0, 16)]
        chv0 = par_v[gi, pl.ds(16, 16)]
        chv1 = par_v[gi, pl.ds(32, 16)]
        chvec = jnp.where(is0, chv0, chv1)
        mx = pa[0]
        my = pa[1]
        mz = pa[2]
        rf = pa[3]
        c0 = pa[4]
        c1 = pa[5]
        c2 = pa[6]
        c3 = pa[7]
        c4 = pa[8]
        c5 = pa[9]
        cx = pa[10]
        cy = pa[11]
        cz = pa[12]
        flatc = pa[13]
        bx = -5.0 - mx
        by = -5.0 - my
        bz = -5.0 - mz
        ch = [chvec[j] for j in range(NCH)]

        def emit_groups(k0, ng, ib, vt):
            for kk in range(ng):
                o = (k0 + kk) * 16
                oxv = offs_v[pl.ds(o, 16)]
                oyv = offs_v[pl.ds(W_PAD + o, 16)]
                ozv = offs_v[pl.ds(2 * W_PAD + o, 16)]
                chbv = offs_v[pl.ds(3 * W_PAD + o, 16)]
                fov = offs_v[pl.ds(4 * W_PAD + o, 16)]
                vx = cx + oxv
                vy = cy + oyv
                vz = cz + ozv
                valid = ((chbv <= rf)
                         & (vx >= 0.0) & (vx <= 49.0)
                         & (vy >= 0.0) & (vy <= 49.0)
                         & (vz >= 0.0) & (vz <= 49.0))
                dx = (vx + 0.5) * 0.2 + bx
                dy = (vy + 0.5) * 0.2 + by
                dz = (vz + 0.5) * 0.2 + bz
                pw = (-0.5 * (c0 * dx * dx + c1 * dy * dy + c2 * dz * dz)
                      - (c3 * dx * dy + c4 * dy * dz + c5 * dx * dz))
                w = jnp.where(valid, jnp.exp(pw), 0.0)
                fl = jnp.where(valid, (flatc + fov).astype(jnp.int32), 0)
                ib[pl.ds(kk * 16, 16)] = fl
                for j in range(NCH):
                    vt[j, pl.ds(kk * 16, 16)] = w * ch[j]

        def fire(ib, vt):
            ds = [pltpu.async_copy(vt.at[j], accs[j].at[ib], sem, add=True)
                  for j in range(8)]

            @pl.when(is0)
            def _():
                pltpu.async_copy(vt.at[8], accs[8].at[ib], sem,
                                 add=True).wait()

            for d in ds:
                d.wait()

        @pl.when(rf == 0.0)
        def _():
            emit_groups(0, 1, ib16, vt16)
            fire(ib16, vt16)

        @pl.when(rf == 1.0)
        def _():
            emit_groups(0, 2, ib32, vt32)
            fire(ib32, vt32)

        @pl.when(rf == 2.0)
        def _():
            emit_groups(0, 8, ib128, vt128)
            fire(ib128, vt128)

        @pl.when(rf == 3.0)
        def _():
            for seg in range(3):
                emit_groups(seg * 8, 8, ib128, vt128)
                fire(ib128, vt128)

        return 0

    def do_chunk(chk, _):
        g0 = sid * G_PER_TILE + chk * CHUNK_G
        pltpu.sync_copy(par_hbm.at[pl.ds(g0, CHUNK_G)], par_v)
        lax.fori_loop(0, CHUNK_G, do_gauss, 0)
        return 0

    lax.fori_loop(0, N_CHUNKS, do_chunk, 0)

    plsc.subcore_barrier()

    for j in range(NCH):
        for z in range(4):
            sl = pl.ds(base_row + z * ZCHUNK, ZCHUNK)
            pltpu.sync_copy(accs[j].at[sl], zbuf)

            @pl.when(is0)
            def _():
                pltpu.sync_copy(zbuf, out0.at[j, sl])

            @pl.when(jnp.logical_not(is0))
            def _():
                pltpu.sync_copy(zbuf, out1.at[j, sl])


@functools.partial(
    pl.kernel,
    out_type=(jax.ShapeDtypeStruct((NCH, NVP), jnp.float32),
              jax.ShapeDtypeStruct((NCH, NVP), jnp.float32)),
    mesh=plsc.VectorSubcoreMesh(core_axis_name="c", subcore_axis_name="s",
                                num_cores=2, num_subcores=16),
    scratch_types=[
        pltpu.VMEM((5 * W_PAD,), jnp.float32),
        pltpu.VMEM((CHUNK_G, PC), jnp.float32),
        pltpu.VMEM((ZCHUNK,), jnp.float32),
        pltpu.VMEM((16,), jnp.int32),
        pltpu.VMEM((32,), jnp.int32),
        pltpu.VMEM((128,), jnp.int32),
        pltpu.VMEM((NCH, 16), jnp.float32),
        pltpu.VMEM((NCH, 32), jnp.float32),
        pltpu.VMEM((NCH, 128), jnp.float32),
        pltpu.SemaphoreType.DMA,
    ] + [pltpu.VMEM_SHARED((NVP,), jnp.float32) for _ in range(NCH)],
    compiler_params=pltpu.CompilerParams(use_tc_tiling_on_sc=False),
)
def _voxelize(par_hbm, offs_hbm, zrow_hbm, out0, out1, *scratch):
    _voxelize_body(par_hbm, offs_hbm, zrow_hbm, out0, out1,
                   *scratch[:10], scratch[10:])


@jax.jit
def kernel(means3d, opacities, cov3D, radii, features):
    n = means3d.shape[0]
    ones1 = jnp.ones((n, 1), jnp.float32)
    zeros7 = jnp.zeros((n, 7), jnp.float32)
    zeros8 = jnp.zeros((n, 8), jnp.float32)
    chv0 = opacities * jnp.concatenate([ones1, features[:, :8], zeros7],
                                       axis=1)
    chv1 = opacities * jnp.concatenate([features[:, 8:], zeros8], axis=1)
    center = jnp.floor((means3d - (-5.0)) / 0.2)
    flatc = (center[:, :1] * 2500.0 + center[:, 1:2] * 50.0
             + center[:, 2:3])
    params = jnp.concatenate(
        [means3d, radii.astype(jnp.float32)[:, None], cov3D,
         center, flatc, jnp.zeros((n, 2), jnp.float32),
         chv0, chv1], axis=1)
    params = jnp.concatenate(
        [params, jnp.zeros((N_PAD - n, PC), jnp.float32)], axis=0)
    offs = jnp.asarray(_OFFSET_PLANES)
    zrow = jnp.zeros((ZCHUNK,), jnp.float32)
    out0, out1 = _voxelize(params, offs, zrow)
    acc = jnp.concatenate([out0[:, :NV].T, out1[:8, :NV].T],
                          axis=1)
    grid_density = acc[:, :1].reshape(GRID, GRID, GRID, 1)
    grid_feats = acc[:, 1:].reshape(GRID, GRID, GRID, 16)
    return grid_density, grid_feats


# speedup vs baseline: 30.1271x; 1.7882x over previous
import functools
import numpy as np
import jax
import jax.numpy as jnp
from jax import lax
from jax.experimental import pallas as pl
from jax.experimental.pallas import tpu as pltpu
from jax.experimental.pallas import tpu_sc as plsc

N_GAUSS = 50000
N_PAD = 50176
GRID = 50
NV = GRID * GRID * GRID
NVP = 126976
ROWS_PER_TILE = NVP // 16
ZCHUNK = ROWS_PER_TILE // 4
NCH = 9
PC = 48
W_PAD = 384
G_PER_TILE = N_PAD // 16
CHUNK_G = 112
N_CHUNKS = G_PER_TILE // CHUNK_G
CLASS_GROUPS = (1, 2, 8, 24)


def _make_offset_planes():
    r = np.arange(-3, 4)
    ox, oy, oz = np.meshgrid(r, r, r, indexing='ij')
    ox, oy, oz = ox.ravel(), oy.ravel(), oz.ravel()
    cheb = np.max(np.abs(np.stack([ox, oy, oz])), axis=0)
    order = np.argsort(cheb, kind='stable')
    ox, oy, oz, cheb = ox[order], oy[order], oz[order], cheb[order]
    flatoff = ox * (GRID * GRID) + oy * GRID + oz
    planes = np.zeros((5, W_PAD), np.float32)
    planes[3, :] = 1e9
    planes[0, :343] = ox
    planes[1, :343] = oy
    planes[2, :343] = oz
    planes[3, :343] = cheb
    planes[4, :343] = flatoff
    return planes.reshape(-1)


_OFFSET_PLANES = _make_offset_planes()


def _voxelize_body(par_hbm, offs_hbm, zrow_hbm, out0, out1,
                   offs_v, par_v, zbuf, ib16, ib32, ib96, ib128,
                   vt16, vt32, vt96, vt128, sem, accs):
    cid = lax.axis_index("c")
    sid = lax.axis_index("s")
    is0 = cid == 0

    pltpu.sync_copy(offs_hbm, offs_v)

    base_row = sid * ROWS_PER_TILE
    pltpu.sync_copy(zrow_hbm, zbuf)
    for j in range(NCH):
        for z in range(4):
            pltpu.sync_copy(
                zbuf, accs[j].at[pl.ds(base_row + z * ZCHUNK, ZCHUNK)])
    plsc.subcore_barrier()

    def do_gauss(gi, _):
        pa = par_v[gi, pl.ds(0, 16)]
        chv0 = par_v[gi, pl.ds(16, 16)]
        chv1 = par_v[gi, pl.ds(32, 16)]
        chvec = jnp.where(is0, chv0, chv1)
        mx = pa[0]
        my = pa[1]
        mz = pa[2]
        rf = pa[3]
        c0 = pa[4]
        c1 = pa[5]
        c2 = pa[6]
        c3 = pa[7]
        c4 = pa[8]
        c5 = pa[9]
        cx = pa[10]
        cy = pa[11]
        cz = pa[12]
        flatc = pa[13]
        bx = -5.0 - mx
        by = -5.0 - my
        bz = -5.0 - mz
        ch = [chvec[j] for j in range(NCH)]

        def emit_groups(k0, ng, ib, vt):
            for kk in range(ng):
                o = (k0 + kk) * 16
                oxv = offs_v[pl.ds(o, 16)]
                oyv = offs_v[pl.ds(W_PAD + o, 16)]
                ozv = offs_v[pl.ds(2 * W_PAD + o, 16)]
                chbv = offs_v[pl.ds(3 * W_PAD + o, 16)]
                fov = offs_v[pl.ds(4 * W_PAD + o, 16)]
                vx = cx + oxv
                vy = cy + oyv
                vz = cz + ozv
                valid = ((chbv <= rf)
                         & (vx >= 0.0) & (vx <= 49.0)
                         & (vy >= 0.0) & (vy <= 49.0)
                         & (vz >= 0.0) & (vz <= 49.0))
                dx = (vx + 0.5) * 0.2 + bx
                dy = (vy + 0.5) * 0.2 + by
                dz = (vz + 0.5) * 0.2 + bz
                pw = (-0.5 * (c0 * dx * dx + c1 * dy * dy + c2 * dz * dz)
                      - (c3 * dx * dy + c4 * dy * dz + c5 * dx * dz))
                w = jnp.where(valid, jnp.exp(pw), 0.0)
                fl = jnp.where(valid, (flatc + fov).astype(jnp.int32), 0)
                ib[pl.ds(kk * 16, 16)] = fl
                for j in range(NCH):
                    vt[j, pl.ds(kk * 16, 16)] = w * ch[j]

        def fire(ib, vt):
            ds = [pltpu.async_copy(vt.at[j], accs[j].at[ib], sem, add=True)
                  for j in range(8)]

            @pl.when(is0)
            def _():
                pltpu.async_copy(vt.at[8], accs[8].at[ib], sem,
                                 add=True).wait()

            for d in ds:
                d.wait()

        @pl.when(rf == 0.0)
        def _():
            emit_groups(0, 1, ib16, vt16)
            fire(ib16, vt16)

        @pl.when(rf == 1.0)
        def _():
            emit_groups(0, 2, ib32, vt32)
            fire(ib32, vt32)

        @pl.when(rf == 2.0)
        def _():
            emit_groups(0, 8, ib128, vt128)
            fire(ib128, vt128)

        @pl.when(rf == 3.0)
        def _():
            for seg in range(2):
                emit_groups(seg * 8, 8, ib128, vt128)
                fire(ib128, vt128)
            emit_groups(16, 6, ib96, vt96)
            fire(ib96, vt96)

        return 0

    def do_chunk(chk, _):
        g0 = sid * G_PER_TILE + chk * CHUNK_G
        pltpu.sync_copy(par_hbm.at[pl.ds(g0, CHUNK_G)], par_v)
        lax.fori_loop(0, CHUNK_G, do_gauss, 0)
        return 0

    lax.fori_loop(0, N_CHUNKS, do_chunk, 0)

    plsc.subcore_barrier()

    for j in range(NCH):
        for z in range(4):
            sl = pl.ds(base_row + z * ZCHUNK, ZCHUNK)
            pltpu.sync_copy(accs[j].at[sl], zbuf)

            @pl.when(is0)
            def _():
                pltpu.sync_copy(zbuf, out0.at[j, sl])

            @pl.when(jnp.logical_not(is0))
            def _():
                pltpu.sync_copy(zbuf, out1.at[j, sl])


@functools.partial(
    pl.kernel,
    out_type=(jax.ShapeDtypeStruct((NCH, NVP), jnp.float32),
              jax.ShapeDtypeStruct((NCH, NVP), jnp.float32)),
    mesh=plsc.VectorSubcoreMesh(core_axis_name="c", subcore_axis_name="s",
                                num_cores=2, num_subcores=16),
    scratch_types=[
        pltpu.VMEM((5 * W_PAD,), jnp.float32),
        pltpu.VMEM((CHUNK_G, PC), jnp.float32),
        pltpu.VMEM((ZCHUNK,), jnp.float32),
        pltpu.VMEM((16,), jnp.int32),
        pltpu.VMEM((32,), jnp.int32),
        pltpu.VMEM((96,), jnp.int32),
        pltpu.VMEM((128,), jnp.int32),
        pltpu.VMEM((NCH, 16), jnp.float32),
        pltpu.VMEM((NCH, 32), jnp.float32),
        pltpu.VMEM((NCH, 96), jnp.float32),
        pltpu.VMEM((NCH, 128), jnp.float32),
        pltpu.SemaphoreType.DMA,
    ] + [pltpu.VMEM_SHARED((NVP,), jnp.float32) for _ in range(NCH)],
    compiler_params=pltpu.CompilerParams(use_tc_tiling_on_sc=False),
)
def _voxelize(par_hbm, offs_hbm, zrow_hbm, out0, out1, *scratch):
    _voxelize_body(par_hbm, offs_hbm, zrow_hbm, out0, out1,
                   *scratch[:12], scratch[12:])


@jax.jit
def kernel(means3d, opacities, cov3D, radii, features):
    n = means3d.shape[0]
    ones1 = jnp.ones((n, 1), jnp.float32)
    zeros7 = jnp.zeros((n, 7), jnp.float32)
    zeros8 = jnp.zeros((n, 8), jnp.float32)
    chv0 = opacities * jnp.concatenate([ones1, features[:, :8], zeros7],
                                       axis=1)
    chv1 = opacities * jnp.concatenate([features[:, 8:], zeros8], axis=1)
    center = jnp.floor((means3d - (-5.0)) / 0.2)
    flatc = (center[:, :1] * 2500.0 + center[:, 1:2] * 50.0
             + center[:, 2:3])
    params = jnp.concatenate(
        [means3d, radii.astype(jnp.float32)[:, None], cov3D,
         center, flatc, jnp.zeros((n, 2), jnp.float32),
         chv0, chv1], axis=1)
    params = jnp.concatenate(
        [params, jnp.zeros((N_PAD - n, PC), jnp.float32)], axis=0)
    offs = jnp.asarray(_OFFSET_PLANES)
    zrow = jnp.zeros((ZCHUNK,), jnp.float32)
    out0, out1 = _voxelize(params, offs, zrow)
    acc = jnp.concatenate([out0[:, :NV].T, out1[:8, :NV].T],
                          axis=1)
    grid_density = acc[:, :1].reshape(GRID, GRID, GRID, 1)
    grid_feats = acc[:, 1:].reshape(GRID, GRID, GRID, 16)
    return grid_density, grid_feats


# speedup vs baseline: 30.3769x; 1.0083x over previous
import functools
import numpy as np
import jax
import jax.numpy as jnp
from jax import lax
from jax.experimental import pallas as pl
from jax.experimental.pallas import tpu as pltpu
from jax.experimental.pallas import tpu_sc as plsc

N_GAUSS = 50000
N_PAD = 50176
GRID = 50
NV = GRID * GRID * GRID
NVP = 126976
ROWS_PER_TILE = NVP // 16
ZCHUNK = ROWS_PER_TILE // 4
NCH = 9
PC = 48
W_PAD = 384
G_PER_TILE = N_PAD // 16
CHUNK_G = 112
N_CHUNKS = G_PER_TILE // CHUNK_G
CLASS_GROUPS = (1, 2, 8, 24)


def _make_offset_planes():
    r = np.arange(-3, 4)
    ox, oy, oz = np.meshgrid(r, r, r, indexing='ij')
    ox, oy, oz = ox.ravel(), oy.ravel(), oz.ravel()
    cheb = np.max(np.abs(np.stack([ox, oy, oz])), axis=0)
    order = np.argsort(cheb, kind='stable')
    ox, oy, oz, cheb = ox[order], oy[order], oz[order], cheb[order]
    flatoff = ox * (GRID * GRID) + oy * GRID + oz
    planes = np.zeros((5, W_PAD), np.float32)
    planes[3, :] = 1e9
    planes[0, :343] = ox
    planes[1, :343] = oy
    planes[2, :343] = oz
    planes[3, :343] = cheb
    planes[4, :343] = flatoff
    return planes.reshape(-1)


_OFFSET_PLANES = _make_offset_planes()


def _voxelize_body(par_hbm, offs_hbm, zrow_hbm, out0, out1,
                   offs_v, par_v, zbuf, ib16, ib32, ib96, ib128,
                   vt16, vt32, vt96, vt128, sem, accs):
    cid = lax.axis_index("c")
    sid = lax.axis_index("s")
    is0 = cid == 0

    pltpu.sync_copy(offs_hbm, offs_v)

    base_row = sid * ROWS_PER_TILE
    pltpu.sync_copy(zrow_hbm, zbuf)
    for j in range(NCH):
        for z in range(4):
            pltpu.sync_copy(
                zbuf, accs[j].at[pl.ds(base_row + z * ZCHUNK, ZCHUNK)])
    plsc.subcore_barrier()

    def do_gauss(gi, _):
        pa = par_v[gi, pl.ds(0, 16)]
        chv0 = par_v[gi, pl.ds(16, 16)]
        chv1 = par_v[gi, pl.ds(32, 16)]
        chvec = jnp.where(is0, chv0, chv1)
        mx = pa[0]
        my = pa[1]
        mz = pa[2]
        rf = pa[3]
        c0 = pa[4]
        c1 = pa[5]
        c2 = pa[6]
        c3 = pa[7]
        c4 = pa[8]
        c5 = pa[9]
        cx = pa[10]
        cy = pa[11]
        cz = pa[12]
        flatc = pa[13]
        bx = -5.0 - mx
        by = -5.0 - my
        bz = -5.0 - mz
        ch = [chvec[j] for j in range(NCH)]

        def emit_groups(k0, ng, ib, vt):
            for kk in range(ng):
                o = (k0 + kk) * 16
                oxv = offs_v[pl.ds(o, 16)]
                oyv = offs_v[pl.ds(W_PAD + o, 16)]
                ozv = offs_v[pl.ds(2 * W_PAD + o, 16)]
                chbv = offs_v[pl.ds(3 * W_PAD + o, 16)]
                fov = offs_v[pl.ds(4 * W_PAD + o, 16)]
                vx = cx + oxv
                vy = cy + oyv
                vz = cz + ozv
                valid = ((chbv <= rf)
                         & (vx >= 0.0) & (vx <= 49.0)
                         & (vy >= 0.0) & (vy <= 49.0)
                         & (vz >= 0.0) & (vz <= 49.0))
                dx = (vx + 0.5) * 0.2 + bx
                dy = (vy + 0.5) * 0.2 + by
                dz = (vz + 0.5) * 0.2 + bz
                pw = (-0.5 * (c0 * dx * dx + c1 * dy * dy + c2 * dz * dz)
                      - (c3 * dx * dy + c4 * dy * dz + c5 * dx * dz))
                w = jnp.where(valid, jnp.exp(pw), 0.0)
                fl = jnp.where(valid, (flatc + fov).astype(jnp.int32), 0)
                ib[pl.ds(kk * 16, 16)] = fl
                for j in range(NCH):
                    vt[j, pl.ds(kk * 16, 16)] = w * ch[j]

        def fire_nowait(ib, vt):
            ds = [pltpu.async_copy(vt.at[j], accs[j].at[ib], sem, add=True)
                  for j in range(8)]

            @pl.when(is0)
            def _():
                pltpu.async_copy(vt.at[8], accs[8].at[ib], sem,
                                 add=True).wait()

            return ds

        def drain(ds):
            for d in ds:
                d.wait()

        def fire(ib, vt):
            drain(fire_nowait(ib, vt))

        @pl.when(rf == 0.0)
        def _():
            emit_groups(0, 1, ib16, vt16)
            fire(ib16, vt16)

        @pl.when(rf == 1.0)
        def _():
            emit_groups(0, 2, ib32, vt32)
            fire(ib32, vt32)

        @pl.when(rf == 2.0)
        def _():
            emit_groups(0, 8, ib128, vt128)
            fire(ib128, vt128)

        @pl.when(rf == 3.0)
        def _():
            emit_groups(0, 8, ib128, vt128)
            ds_a = fire_nowait(ib128, vt128)
            emit_groups(16, 6, ib96, vt96)
            ds_b = fire_nowait(ib96, vt96)
            drain(ds_a)
            emit_groups(8, 8, ib128, vt128)
            ds_c = fire_nowait(ib128, vt128)
            drain(ds_b)
            drain(ds_c)

        return 0

    def do_chunk(chk, _):
        g0 = sid * G_PER_TILE + chk * CHUNK_G
        pltpu.sync_copy(par_hbm.at[pl.ds(g0, CHUNK_G)], par_v)
        lax.fori_loop(0, CHUNK_G, do_gauss, 0)
        return 0

    lax.fori_loop(0, N_CHUNKS, do_chunk, 0)

    plsc.subcore_barrier()

    for j in range(NCH):
        for z in range(4):
            sl = pl.ds(base_row + z * ZCHUNK, ZCHUNK)
            pltpu.sync_copy(accs[j].at[sl], zbuf)

            @pl.when(is0)
            def _():
                pltpu.sync_copy(zbuf, out0.at[j, sl])

            @pl.when(jnp.logical_not(is0))
            def _():
                pltpu.sync_copy(zbuf, out1.at[j, sl])


@functools.partial(
    pl.kernel,
    out_type=(jax.ShapeDtypeStruct((NCH, NVP), jnp.float32),
              jax.ShapeDtypeStruct((NCH, NVP), jnp.float32)),
    mesh=plsc.VectorSubcoreMesh(core_axis_name="c", subcore_axis_name="s",
                                num_cores=2, num_subcores=16),
    scratch_types=[
        pltpu.VMEM((5 * W_PAD,), jnp.float32),
        pltpu.VMEM((CHUNK_G, PC), jnp.float32),
        pltpu.VMEM((ZCHUNK,), jnp.float32),
        pltpu.VMEM((16,), jnp.int32),
        pltpu.VMEM((32,), jnp.int32),
        pltpu.VMEM((96,), jnp.int32),
        pltpu.VMEM((128,), jnp.int32),
        pltpu.VMEM((NCH, 16), jnp.float32),
        pltpu.VMEM((NCH, 32), jnp.float32),
        pltpu.VMEM((NCH, 96), jnp.float32),
        pltpu.VMEM((NCH, 128), jnp.float32),
        pltpu.SemaphoreType.DMA,
    ] + [pltpu.VMEM_SHARED((NVP,), jnp.float32) for _ in range(NCH)],
    compiler_params=pltpu.CompilerParams(use_tc_tiling_on_sc=False),
)
def _voxelize(par_hbm, offs_hbm, zrow_hbm, out0, out1, *scratch):
    _voxelize_body(par_hbm, offs_hbm, zrow_hbm, out0, out1,
                   *scratch[:12], scratch[12:])


@jax.jit
def kernel(means3d, opacities, cov3D, radii, features):
    n = means3d.shape[0]
    ones1 = jnp.ones((n, 1), jnp.float32)
    zeros7 = jnp.zeros((n, 7), jnp.float32)
    zeros8 = jnp.zeros((n, 8), jnp.float32)
    chv0 = opacities * jnp.concatenate([ones1, features[:, :8], zeros7],
                                       axis=1)
    chv1 = opacities * jnp.concatenate([features[:, 8:], zeros8], axis=1)
    center = jnp.floor((means3d - (-5.0)) / 0.2)
    flatc = (center[:, :1] * 2500.0 + center[:, 1:2] * 50.0
             + center[:, 2:3])
    params = jnp.concatenate(
        [means3d, radii.astype(jnp.float32)[:, None], cov3D,
         center, flatc, jnp.zeros((n, 2), jnp.float32),
         chv0, chv1], axis=1)
    params = jnp.concatenate(
        [params, jnp.zeros((N_PAD - n, PC), jnp.float32)], axis=0)
    offs = jnp.asarray(_OFFSET_PLANES)
    zrow = jnp.zeros((ZCHUNK,), jnp.float32)
    out0, out1 = _voxelize(params, offs, zrow)
    acc = jnp.concatenate([out0[:, :NV].T, out1[:8, :NV].T],
                          axis=1)
    grid_density = acc[:, :1].reshape(GRID, GRID, GRID, 1)
    grid_feats = acc[:, 1:].reshape(GRID, GRID, GRID, 16)
    return grid_density, grid_feats
